# Initial kernel scaffold; baseline (speedup 1.0000x reference)
import jax
import jax.numpy as jnp
from jax.experimental import pallas as pl


def kernel(x, edge_index, edge_attr, w0, root0, b0, w1, root1, b1, w2, root2, b2, w3, root3, b3, w4, root4, b4, w5, root5, b5, lin_w, lin_b):
    raise NotImplementedError("write your pallas kernel here")


# speedup vs baseline: 1.0071x; 1.0071x over previous
import functools

import jax
import jax.numpy as jnp
from jax import lax
from jax.experimental import pallas as pl
from jax.experimental.pallas import tpu as pltpu

KS = 5
DIM = 3
K = KS ** DIM
KPAD = 128
N = 2048
E = 8192
FEAT = 64
IN0P = 16
LAYERS = 6

EBLK = 1024
NBLK = 256

_HI = lax.Precision.HIGHEST


def _prep_body(ei_ref, attr_ref, B_ref, G_ref, D_ref, deg_ref):
    i = pl.program_id(0)
    eiT = jnp.transpose(ei_ref[...])
    srcc = eiT[:, 0:1]
    dst_row = ei_ref[1:2, :]

    p = attr_ref[...] * (KS - 1.0)
    lo = jnp.floor(p)
    frac = p - lo
    lo_i = jnp.clip(lo.astype(jnp.int32), 0, KS - 1)

    bidx = lax.broadcasted_iota(jnp.int32, (8, 1), 0)
    basis8 = jnp.ones((8, EBLK), jnp.float32)
    widx8 = jnp.zeros((8, EBLK), jnp.int32)
    for d in range(DIM):
        bi = (bidx >> d) & 1
        bf = bi.astype(jnp.float32)
        f = frac[d:d + 1, :]
        basis8 = basis8 * (bf * f + (1.0 - bf) * (1.0 - f))
        ii = jnp.clip(lo_i[d:d + 1, :] + bi, 0, KS - 1)
        widx8 = widx8 * KS + ii

    basisT = jnp.transpose(basis8)
    widxT = jnp.transpose(widx8)

    kio = lax.broadcasted_iota(jnp.int32, (EBLK, KPAD), 1)
    B = jnp.zeros((EBLK, KPAD), jnp.float32)
    for b in range(8):
        B = B + jnp.where(widxT[:, b:b + 1] == kio,
                          basisT[:, b:b + 1], 0.0)
    B_ref[...] = B

    nio = lax.broadcasted_iota(jnp.int32, (EBLK, N), 1)
    G_ref[...] = (srcc == nio).astype(jnp.float32)

    nio0 = lax.broadcasted_iota(jnp.int32, (N, EBLK), 0)
    Dblk = (dst_row == nio0).astype(jnp.float32)
    D_ref[...] = Dblk

    part = jnp.sum(Dblk, axis=1, keepdims=True)

    @pl.when(i == 0)
    def _():
        deg_ref[...] = jnp.zeros_like(deg_ref)

    deg_ref[...] += part


def _msg_body(in_ch, G_ref, B_ref, h_ref, W_ref, y_ref):
    xs = jnp.dot(G_ref[...], h_ref[...],
                 preferred_element_type=jnp.float32, precision=_HI)
    acc = jnp.zeros((EBLK, FEAT), jnp.float32)
    for g in range(KPAD // 8):
        A_g = jnp.concatenate(
            [B_ref[:, g * 8 + j:g * 8 + j + 1] * xs for j in range(8)],
            axis=1)
        Wg = W_ref[g * 8 * in_ch:(g + 1) * 8 * in_ch, :]
        acc += jnp.dot(A_g, Wg, preferred_element_type=jnp.float32,
                       precision=_HI)
    y_ref[...] = acc


def _reduce_body(D_ref, y_ref, h_ref, deg_ref, R_ref, b_ref, o_ref):
    msg = jnp.dot(D_ref[...], y_ref[...],
                  preferred_element_type=jnp.float32, precision=_HI)
    invd = 1.0 / jnp.maximum(deg_ref[...], 1.0)
    root = jnp.dot(h_ref[...], R_ref[...],
                   preferred_element_type=jnp.float32, precision=_HI)
    o_ref[...] = jnp.maximum(msg * invd + root + b_ref[...], 0.0)


def _final_body(x_ref, h1, h2, h3, h4, h5, h6,
                lw0, lw1, lw2, lw3, lw4, lw5, lw6, lb_ref, o_ref):
    hs = [x_ref, h1, h2, h3, h4, h5, h6]
    lws = [lw0, lw1, lw2, lw3, lw4, lw5, lw6]
    acc = jnp.zeros((FEAT, N), jnp.float32)
    for h_ref, lw_ref in zip(hs, lws):
        acc += lax.dot_general(lw_ref[...], h_ref[...],
                               (((0,), (1,)), ((), ())),
                               preferred_element_type=jnp.float32,
                               precision=_HI)
    o_ref[...] = acc + lb_ref[...]


def kernel(x, edge_index, edge_attr, w0, root0, b0, w1, root1, b1,
           w2, root2, b2, w3, root3, b3, w4, root4, b4, w5, root5, b5,
           lin_w, lin_b):
    f32 = jnp.float32
    x16 = jnp.pad(x.astype(f32), ((0, 0), (0, IN0P - x.shape[1])))
    attr_t = edge_attr.astype(f32).T
    ei = edge_index.astype(jnp.int32)

    def flat_w(w, in_p):
        kp = jnp.zeros((KPAD, in_p, FEAT), f32)
        kp = kp.at[:K, :w.shape[1], :].set(w.astype(f32))
        return kp.reshape(KPAD * in_p, FEAT)

    Ws = [flat_w(w0, IN0P)] + [flat_w(w, FEAT)
                               for w in (w1, w2, w3, w4, w5)]
    r0p = jnp.zeros((IN0P, FEAT), f32).at[:root0.shape[0], :].set(
        root0.astype(f32))
    Rs = [r0p] + [r.astype(f32) for r in (root1, root2, root3, root4, root5)]
    Bs = [b.astype(f32).reshape(1, FEAT)
          for b in (b0, b1, b2, b3, b4, b5)]

    lw0p = jnp.zeros((IN0P, FEAT), f32).at[:x.shape[1], :].set(
        lin_w[:x.shape[1], :].astype(f32))
    lws = [lw0p] + [lin_w[x.shape[1] + l * FEAT:
                          x.shape[1] + (l + 1) * FEAT, :].astype(f32)
                    for l in range(LAYERS)]
    lb_col = lin_b.astype(f32).reshape(FEAT, 1)

    nblks = E // EBLK
    B_mat, G, D, deg = pl.pallas_call(
        _prep_body,
        grid=(nblks,),
        in_specs=[
            pl.BlockSpec((2, EBLK), lambda i: (0, i)),
            pl.BlockSpec((3, EBLK), lambda i: (0, i)),
        ],
        out_specs=[
            pl.BlockSpec((EBLK, KPAD), lambda i: (i, 0)),
            pl.BlockSpec((EBLK, N), lambda i: (i, 0)),
            pl.BlockSpec((N, EBLK), lambda i: (0, i)),
            pl.BlockSpec((N, 1), lambda i: (0, 0)),
        ],
        out_shape=[
            jax.ShapeDtypeStruct((E, KPAD), f32),
            jax.ShapeDtypeStruct((E, N), f32),
            jax.ShapeDtypeStruct((N, E), f32),
            jax.ShapeDtypeStruct((N, 1), f32),
        ],
        compiler_params=pltpu.CompilerParams(
            vmem_limit_bytes=100 * 1024 * 1024),
    )(ei, attr_t)

    h = x16
    feats = []
    for l in range(LAYERS):
        in_ch = IN0P if l == 0 else FEAT
        y = pl.pallas_call(
            functools.partial(_msg_body, in_ch),
            grid=(nblks,),
            in_specs=[
                pl.BlockSpec((EBLK, N), lambda i: (i, 0)),
                pl.BlockSpec((EBLK, KPAD), lambda i: (i, 0)),
                pl.BlockSpec((N, in_ch), lambda i: (0, 0)),
                pl.BlockSpec((KPAD * in_ch, FEAT), lambda i: (0, 0)),
            ],
            out_specs=pl.BlockSpec((EBLK, FEAT), lambda i: (i, 0)),
            out_shape=jax.ShapeDtypeStruct((E, FEAT), f32),
            compiler_params=pltpu.CompilerParams(
                vmem_limit_bytes=100 * 1024 * 1024),
        )(G, B_mat, h, Ws[l])

        h = pl.pallas_call(
            _reduce_body,
            grid=(N // NBLK,),
            in_specs=[
                pl.BlockSpec((NBLK, E), lambda i: (i, 0)),
                pl.BlockSpec((E, FEAT), lambda i: (0, 0)),
                pl.BlockSpec((NBLK, in_ch), lambda i: (i, 0)),
                pl.BlockSpec((NBLK, 1), lambda i: (i, 0)),
                pl.BlockSpec((in_ch, FEAT), lambda i: (0, 0)),
                pl.BlockSpec((1, FEAT), lambda i: (0, 0)),
            ],
            out_specs=pl.BlockSpec((NBLK, FEAT), lambda i: (i, 0)),
            out_shape=jax.ShapeDtypeStruct((N, FEAT), f32),
            compiler_params=pltpu.CompilerParams(
                vmem_limit_bytes=100 * 1024 * 1024),
        )(D, y, h, deg, Rs[l], Bs[l])
        feats.append(h)

    out = pl.pallas_call(
        _final_body,
        in_specs=[pl.BlockSpec((N, IN0P), lambda: (0, 0))]
        + [pl.BlockSpec((N, FEAT), lambda: (0, 0))] * LAYERS
        + [pl.BlockSpec((IN0P, FEAT), lambda: (0, 0))]
        + [pl.BlockSpec((FEAT, FEAT), lambda: (0, 0))] * LAYERS
        + [pl.BlockSpec((FEAT, 1), lambda: (0, 0))],
        out_specs=pl.BlockSpec((FEAT, N), lambda: (0, 0)),
        out_shape=jax.ShapeDtypeStruct((FEAT, N), f32),
        compiler_params=pltpu.CompilerParams(
            vmem_limit_bytes=100 * 1024 * 1024),
    )(x16, *feats, *lws, lb_col)
    return out


# speedup vs baseline: 2.5395x; 2.5217x over previous
import functools

import jax
import jax.numpy as jnp
from jax import lax
from jax.experimental import pallas as pl
from jax.experimental.pallas import tpu as pltpu

KS = 5
DIM = 3
K = KS ** DIM
KPAD = 128
N = 2048
E = 8192
FEAT = 64
IN0P = 16
LAYERS = 6

EBLK = 1024
NBLK = 256

_HI = lax.Precision.DEFAULT


def _prep_body(ei_ref, attr_ref, B_ref, G_ref, D_ref, deg_ref):
    i = pl.program_id(0)
    eiT = jnp.transpose(ei_ref[...])
    srcc = eiT[:, 0:1]
    dst_row = ei_ref[1:2, :]

    p = attr_ref[...] * (KS - 1.0)
    lo = jnp.floor(p)
    frac = p - lo
    lo_i = jnp.clip(lo.astype(jnp.int32), 0, KS - 1)

    bidx = lax.broadcasted_iota(jnp.int32, (8, 1), 0)
    basis8 = jnp.ones((8, EBLK), jnp.float32)
    widx8 = jnp.zeros((8, EBLK), jnp.int32)
    for d in range(DIM):
        bi = (bidx >> d) & 1
        bf = bi.astype(jnp.float32)
        f = frac[d:d + 1, :]
        basis8 = basis8 * (bf * f + (1.0 - bf) * (1.0 - f))
        ii = jnp.clip(lo_i[d:d + 1, :] + bi, 0, KS - 1)
        widx8 = widx8 * KS + ii

    basisT = jnp.transpose(basis8)
    widxT = jnp.transpose(widx8)

    kio = lax.broadcasted_iota(jnp.int32, (EBLK, KPAD), 1)
    B = jnp.zeros((EBLK, KPAD), jnp.float32)
    for b in range(8):
        B = B + jnp.where(widxT[:, b:b + 1] == kio,
                          basisT[:, b:b + 1], 0.0)
    B_ref[...] = B

    nio = lax.broadcasted_iota(jnp.int32, (EBLK, N), 1)
    G_ref[...] = (srcc == nio).astype(jnp.float32)

    nio0 = lax.broadcasted_iota(jnp.int32, (N, EBLK), 0)
    Dblk = (dst_row == nio0).astype(jnp.float32)
    D_ref[...] = Dblk

    part = jnp.sum(Dblk, axis=1, keepdims=True)

    @pl.when(i == 0)
    def _():
        deg_ref[...] = jnp.zeros_like(deg_ref)

    deg_ref[...] += part


def _msg_body(in_ch, G_ref, B_ref, h_ref, W_ref, y_ref):
    xs = jnp.dot(G_ref[...], h_ref[...],
                 preferred_element_type=jnp.float32, precision=_HI)
    acc = jnp.zeros((EBLK, FEAT), jnp.float32)
    for g in range(KPAD // 8):
        A_g = jnp.concatenate(
            [B_ref[:, g * 8 + j:g * 8 + j + 1] * xs for j in range(8)],
            axis=1)
        Wg = W_ref[g * 8 * in_ch:(g + 1) * 8 * in_ch, :]
        acc += jnp.dot(A_g, Wg, preferred_element_type=jnp.float32,
                       precision=_HI)
    y_ref[...] = acc


def _reduce_body(D_ref, y_ref, h_ref, deg_ref, R_ref, b_ref, o_ref):
    msg = jnp.dot(D_ref[...], y_ref[...],
                  preferred_element_type=jnp.float32, precision=_HI)
    invd = 1.0 / jnp.maximum(deg_ref[...], 1.0)
    root = jnp.dot(h_ref[...], R_ref[...],
                   preferred_element_type=jnp.float32, precision=_HI)
    o_ref[...] = jnp.maximum(msg * invd + root + b_ref[...], 0.0)


def _final_body(x_ref, h1, h2, h3, h4, h5, h6,
                lw0, lw1, lw2, lw3, lw4, lw5, lw6, lb_ref, o_ref):
    hs = [x_ref, h1, h2, h3, h4, h5, h6]
    lws = [lw0, lw1, lw2, lw3, lw4, lw5, lw6]
    acc = jnp.zeros((FEAT, N), jnp.float32)
    for h_ref, lw_ref in zip(hs, lws):
        acc += lax.dot_general(lw_ref[...], h_ref[...],
                               (((0,), (1,)), ((), ())),
                               preferred_element_type=jnp.float32,
                               precision=_HI)
    o_ref[...] = acc + lb_ref[...]


def kernel(x, edge_index, edge_attr, w0, root0, b0, w1, root1, b1,
           w2, root2, b2, w3, root3, b3, w4, root4, b4, w5, root5, b5,
           lin_w, lin_b):
    f32 = jnp.float32
    x16 = jnp.pad(x.astype(f32), ((0, 0), (0, IN0P - x.shape[1])))
    attr_t = edge_attr.astype(f32).T
    ei = edge_index.astype(jnp.int32)

    def flat_w(w, in_p):
        kp = jnp.zeros((KPAD, in_p, FEAT), f32)
        kp = kp.at[:K, :w.shape[1], :].set(w.astype(f32))
        return kp.reshape(KPAD * in_p, FEAT)

    Ws = [flat_w(w0, IN0P)] + [flat_w(w, FEAT)
                               for w in (w1, w2, w3, w4, w5)]
    r0p = jnp.zeros((IN0P, FEAT), f32).at[:root0.shape[0], :].set(
        root0.astype(f32))
    Rs = [r0p] + [r.astype(f32) for r in (root1, root2, root3, root4, root5)]
    Bs = [b.astype(f32).reshape(1, FEAT)
          for b in (b0, b1, b2, b3, b4, b5)]

    lw0p = jnp.zeros((IN0P, FEAT), f32).at[:x.shape[1], :].set(
        lin_w[:x.shape[1], :].astype(f32))
    lws = [lw0p] + [lin_w[x.shape[1] + l * FEAT:
                          x.shape[1] + (l + 1) * FEAT, :].astype(f32)
                    for l in range(LAYERS)]
    lb_col = lin_b.astype(f32).reshape(FEAT, 1)

    nblks = E // EBLK
    B_mat, G, D, deg = pl.pallas_call(
        _prep_body,
        grid=(nblks,),
        in_specs=[
            pl.BlockSpec((2, EBLK), lambda i: (0, i)),
            pl.BlockSpec((3, EBLK), lambda i: (0, i)),
        ],
        out_specs=[
            pl.BlockSpec((EBLK, KPAD), lambda i: (i, 0)),
            pl.BlockSpec((EBLK, N), lambda i: (i, 0)),
            pl.BlockSpec((N, EBLK), lambda i: (0, i)),
            pl.BlockSpec((N, 1), lambda i: (0, 0)),
        ],
        out_shape=[
            jax.ShapeDtypeStruct((E, KPAD), f32),
            jax.ShapeDtypeStruct((E, N), f32),
            jax.ShapeDtypeStruct((N, E), f32),
            jax.ShapeDtypeStruct((N, 1), f32),
        ],
        compiler_params=pltpu.CompilerParams(
            vmem_limit_bytes=100 * 1024 * 1024),
    )(ei, attr_t)

    h = x16
    feats = []
    for l in range(LAYERS):
        in_ch = IN0P if l == 0 else FEAT
        y = pl.pallas_call(
            functools.partial(_msg_body, in_ch),
            grid=(nblks,),
            in_specs=[
                pl.BlockSpec((EBLK, N), lambda i: (i, 0)),
                pl.BlockSpec((EBLK, KPAD), lambda i: (i, 0)),
                pl.BlockSpec((N, in_ch), lambda i: (0, 0)),
                pl.BlockSpec((KPAD * in_ch, FEAT), lambda i: (0, 0)),
            ],
            out_specs=pl.BlockSpec((EBLK, FEAT), lambda i: (i, 0)),
            out_shape=jax.ShapeDtypeStruct((E, FEAT), f32),
            compiler_params=pltpu.CompilerParams(
                vmem_limit_bytes=100 * 1024 * 1024),
        )(G, B_mat, h, Ws[l])

        h = pl.pallas_call(
            _reduce_body,
            grid=(N // NBLK,),
            in_specs=[
                pl.BlockSpec((NBLK, E), lambda i: (i, 0)),
                pl.BlockSpec((E, FEAT), lambda i: (0, 0)),
                pl.BlockSpec((NBLK, in_ch), lambda i: (i, 0)),
                pl.BlockSpec((NBLK, 1), lambda i: (i, 0)),
                pl.BlockSpec((in_ch, FEAT), lambda i: (0, 0)),
                pl.BlockSpec((1, FEAT), lambda i: (0, 0)),
            ],
            out_specs=pl.BlockSpec((NBLK, FEAT), lambda i: (i, 0)),
            out_shape=jax.ShapeDtypeStruct((N, FEAT), f32),
            compiler_params=pltpu.CompilerParams(
                vmem_limit_bytes=100 * 1024 * 1024),
        )(D, y, h, deg, Rs[l], Bs[l])
        feats.append(h)

    out = pl.pallas_call(
        _final_body,
        in_specs=[pl.BlockSpec((N, IN0P), lambda: (0, 0))]
        + [pl.BlockSpec((N, FEAT), lambda: (0, 0))] * LAYERS
        + [pl.BlockSpec((IN0P, FEAT), lambda: (0, 0))]
        + [pl.BlockSpec((FEAT, FEAT), lambda: (0, 0))] * LAYERS
        + [pl.BlockSpec((FEAT, 1), lambda: (0, 0))],
        out_specs=pl.BlockSpec((FEAT, N), lambda: (0, 0)),
        out_shape=jax.ShapeDtypeStruct((FEAT, N), f32),
        compiler_params=pltpu.CompilerParams(
            vmem_limit_bytes=100 * 1024 * 1024),
    )(x16, *feats, *lws, lb_col)
    return out
